# baseline (speedup 1.0000x reference)
import jax
import jax.numpy as jnp
from jax.experimental import pallas as pl
from jax.experimental.pallas import tpu as pltpu


def _vq_body(x_ref, dict_ref, q_ref, loss_ref, dt_ref, nrm_ref):
    i = pl.program_id(0)
    k = dict_ref.shape[1]

    @pl.when(i == 0)
    def _prep():
        d0 = dict_ref[...]
        dt_ref[...] = d0.T
        nrm_ref[...] = jnp.sum(d0 * d0, axis=0, keepdims=True)

    x = x_ref[...]
    d = dict_ref[...]
    dt = dt_ref[...]
    lane = jax.lax.broadcasted_iota(jnp.int32, (1, k), 1).astype(jnp.float32)
    scores2 = jnp.dot(x + x, d, preferred_element_type=jnp.float32)
    norms = nrm_ref[...]
    row_norms = jnp.sum(x * x, axis=1, keepdims=True)
    dist = (row_norms + norms) - scores2
    m = jnp.min(dist, axis=1, keepdims=True)
    idx = jnp.min(jnp.where(dist == m, lane, float(k)), axis=1, keepdims=True)
    onehot = (lane == idx).astype(jnp.float32)
    q = jnp.dot(onehot, dt, preferred_element_type=jnp.float32)
    q_ref[...] = q
    part = jnp.sum(m).reshape(1, 1)

    @pl.when(i == 0)
    def _init():
        loss_ref[...] = jnp.zeros_like(loss_ref)

    loss_ref[...] += part


def kernel(x, dictionary):
    beta = 0.25
    img_dims = x.shape
    d_dim, k_dim = dictionary.shape
    flat = x.reshape(-1, d_dim)
    n = flat.shape[0]
    blk = 1024

    q, loss_sum = pl.pallas_call(
        _vq_body,
        grid=(n // blk,),
        in_specs=[
            pl.BlockSpec((blk, d_dim), lambda i: (i, 0)),
            pl.BlockSpec((d_dim, k_dim), lambda i: (0, 0)),
        ],
        out_specs=[
            pl.BlockSpec((blk, d_dim), lambda i: (i, 0)),
            pl.BlockSpec((1, 1), lambda i: (0, 0)),
        ],
        out_shape=[
            jax.ShapeDtypeStruct((n, d_dim), jnp.float32),
            jax.ShapeDtypeStruct((1, 1), jnp.float32),
        ],
        scratch_shapes=[
            pltpu.VMEM((k_dim, d_dim), jnp.float32),
            pltpu.VMEM((1, k_dim), jnp.float32),
        ],
    )(flat, dictionary)

    q = q.reshape(img_dims)
    loss = (1.0 + beta) * loss_sum[0, 0] / x.size
    return q, loss


# speedup vs baseline: 1.0575x; 1.0575x over previous
import jax
import jax.numpy as jnp
from jax.experimental import pallas as pl
from jax.experimental.pallas import tpu as pltpu


def _vq_body(x_ref, dict_ref, q_ref, loss_ref, dt_ref, nrm_ref):
    i = pl.program_id(0)
    k = dict_ref.shape[1]

    @pl.when(i == 0)
    def _prep():
        d0 = dict_ref[...]
        dt_ref[...] = d0.T
        nrm_ref[...] = jnp.sum(d0 * d0, axis=0, keepdims=True)

    x = x_ref[...]
    d = dict_ref[...]
    dt = dt_ref[...]
    lane = jax.lax.broadcasted_iota(jnp.int32, (1, k), 1).astype(jnp.float32)
    scores2 = jnp.dot(x + x, d, preferred_element_type=jnp.float32)
    norms = nrm_ref[...]
    row_norms = jnp.sum(x * x, axis=1, keepdims=True)
    dist = (row_norms + norms) - scores2
    m = jnp.min(dist, axis=1, keepdims=True)
    idx = jnp.min(jnp.where(dist == m, lane, float(k)), axis=1, keepdims=True)
    onehot = (lane == idx).astype(jnp.float32)
    q = jnp.dot(onehot, dt, preferred_element_type=jnp.float32)
    q_ref[...] = q
    part = jnp.sum(m).reshape(1, 1)

    @pl.when(i == 0)
    def _init():
        loss_ref[...] = jnp.zeros_like(loss_ref)

    loss_ref[...] += part


def kernel(x, dictionary):
    beta = 0.25
    img_dims = x.shape
    d_dim, k_dim = dictionary.shape
    flat = x.reshape(-1, d_dim)
    n = flat.shape[0]
    blk = 2048

    q, loss_sum = pl.pallas_call(
        _vq_body,
        grid=(n // blk,),
        in_specs=[
            pl.BlockSpec((blk, d_dim), lambda i: (i, 0)),
            pl.BlockSpec((d_dim, k_dim), lambda i: (0, 0)),
        ],
        out_specs=[
            pl.BlockSpec((blk, d_dim), lambda i: (i, 0)),
            pl.BlockSpec((1, 1), lambda i: (0, 0)),
        ],
        out_shape=[
            jax.ShapeDtypeStruct((n, d_dim), jnp.float32),
            jax.ShapeDtypeStruct((1, 1), jnp.float32),
        ],
        scratch_shapes=[
            pltpu.VMEM((k_dim, d_dim), jnp.float32),
            pltpu.VMEM((1, k_dim), jnp.float32),
        ],
    )(flat, dictionary)

    q = q.reshape(img_dims)
    loss = (1.0 + beta) * loss_sum[0, 0] / x.size
    return q, loss


# speedup vs baseline: 1.0752x; 1.0168x over previous
import jax
import jax.numpy as jnp
from jax.experimental import pallas as pl
from jax.experimental.pallas import tpu as pltpu


def _vq_body(x_ref, dict_ref, q_ref, loss_ref, dt_ref, nrm_ref):
    i = pl.program_id(0)
    k = dict_ref.shape[1]

    @pl.when(i == 0)
    def _prep():
        d0 = dict_ref[...]
        dt_ref[...] = d0.T
        nrm_ref[...] = jnp.sum(d0 * d0, axis=0, keepdims=True)

    x = x_ref[...]
    d = dict_ref[...]
    dt = dt_ref[...]
    lane = jax.lax.broadcasted_iota(jnp.int32, (1, k), 1).astype(jnp.float32)
    scores2 = jnp.dot(x + x, d, preferred_element_type=jnp.float32)
    norms = nrm_ref[...]
    row_norms = jnp.sum(x * x, axis=1, keepdims=True)
    dist = (row_norms + norms) - scores2
    m = jnp.min(dist, axis=1, keepdims=True)
    idx = jnp.min(jnp.where(dist == m, lane, float(k)), axis=1, keepdims=True)
    onehot = (lane == idx).astype(jnp.float32)
    q = jnp.dot(onehot, dt, preferred_element_type=jnp.float32)
    q_ref[...] = q
    part = jnp.sum(m).reshape(1, 1)

    @pl.when(i == 0)
    def _init():
        loss_ref[...] = jnp.zeros_like(loss_ref)

    loss_ref[...] += part


def kernel(x, dictionary):
    beta = 0.25
    img_dims = x.shape
    d_dim, k_dim = dictionary.shape
    flat = x.reshape(-1, d_dim)
    n = flat.shape[0]
    blk = 4096

    q, loss_sum = pl.pallas_call(
        _vq_body,
        grid=(n // blk,),
        in_specs=[
            pl.BlockSpec((blk, d_dim), lambda i: (i, 0)),
            pl.BlockSpec((d_dim, k_dim), lambda i: (0, 0)),
        ],
        out_specs=[
            pl.BlockSpec((blk, d_dim), lambda i: (i, 0)),
            pl.BlockSpec((1, 1), lambda i: (0, 0)),
        ],
        out_shape=[
            jax.ShapeDtypeStruct((n, d_dim), jnp.float32),
            jax.ShapeDtypeStruct((1, 1), jnp.float32),
        ],
        scratch_shapes=[
            pltpu.VMEM((k_dim, d_dim), jnp.float32),
            pltpu.VMEM((1, k_dim), jnp.float32),
        ],
    )(flat, dictionary)

    q = q.reshape(img_dims)
    loss = (1.0 + beta) * loss_sum[0, 0] / x.size
    return q, loss


# speedup vs baseline: 1.0832x; 1.0074x over previous
import jax
import jax.numpy as jnp
from jax.experimental import pallas as pl


def _vq_body(x_ref, dict_ref, q_ref, loss_ref):
    i = pl.program_id(0)
    x = x_ref[...]
    d = dict_ref[...]
    dt = d.T
    k = d.shape[1]
    lane = jax.lax.broadcasted_iota(jnp.int32, (1, k), 1).astype(jnp.float32)
    scores2 = jnp.dot(x + x, d, preferred_element_type=jnp.float32)
    norms = jnp.sum(d * d, axis=0, keepdims=True)
    row_norms = jnp.sum(x * x, axis=1, keepdims=True)
    dist = (row_norms + norms) - scores2
    m = jnp.min(dist, axis=1, keepdims=True)
    idx = jnp.min(jnp.where(dist == m, lane, float(k)), axis=1, keepdims=True)
    onehot = (lane == idx).astype(jnp.float32)
    q = jnp.dot(onehot, dt, preferred_element_type=jnp.float32)
    q_ref[...] = q
    part = jnp.sum(m).reshape(1, 1)

    @pl.when(i == 0)
    def _init():
        loss_ref[...] = jnp.zeros_like(loss_ref)

    loss_ref[...] += part


def kernel(x, dictionary):
    beta = 0.25
    img_dims = x.shape
    d_dim, k_dim = dictionary.shape
    flat = x.reshape(-1, d_dim)
    n = flat.shape[0]
    blk = 4096

    q, loss_sum = pl.pallas_call(
        _vq_body,
        grid=(n // blk,),
        in_specs=[
            pl.BlockSpec((blk, d_dim), lambda i: (i, 0)),
            pl.BlockSpec((d_dim, k_dim), lambda i: (0, 0)),
        ],
        out_specs=[
            pl.BlockSpec((blk, d_dim), lambda i: (i, 0)),
            pl.BlockSpec((1, 1), lambda i: (0, 0)),
        ],
        out_shape=[
            jax.ShapeDtypeStruct((n, d_dim), jnp.float32),
            jax.ShapeDtypeStruct((1, 1), jnp.float32),
        ],
    )(flat, dictionary)

    q = q.reshape(img_dims)
    loss = (1.0 + beta) * loss_sum[0, 0] / x.size
    return q, loss
